# baseline (speedup 1.0000x reference)
import functools

import jax
import jax.numpy as jnp
from jax import lax
from jax.experimental import pallas as pl
from jax.experimental.pallas import tpu as pltpu
from jax.experimental.pallas import tpu_sc as plsc

_VOCAB_TILE = 2048


def _make_sc_gather(V, D, B):
    info = plsc.get_sparse_core_info()
    NC, NS = info.num_cores, info.num_subcores
    NW = NC * NS
    assert D % info.num_lanes == 0 and B % (8 * NW) == 0
    b_per_w = B // NW
    mesh = plsc.VectorSubcoreMesh(core_axis_name="c", subcore_axis_name="s")

    @functools.partial(
        pl.kernel,
        mesh=mesh,
        out_type=jax.ShapeDtypeStruct((B, D), jnp.float32),
        scratch_types=[
            pltpu.VMEM((b_per_w,), jnp.int32),
            pltpu.VMEM((b_per_w, D), jnp.float32),
            pltpu.SemaphoreType.DMA,
        ],
        compiler_params=pltpu.CompilerParams(use_tc_tiling_on_sc=False),
    )
    def gather(table_hbm, idx_hbm, out_hbm, idx_v, rows_v, sem):
        wid = lax.axis_index("s") * NC + lax.axis_index("c")
        base = wid * b_per_w
        pltpu.sync_copy(idx_hbm.at[pl.ds(base, b_per_w)], idx_v)
        pltpu.async_copy(table_hbm.at[idx_v], rows_v, sem).wait()
        pltpu.sync_copy(rows_v, out_hbm.at[pl.ds(base, b_per_w)])

    return gather


def _mlp_body(emb_ref, w1_ref, b1_ref, w2_ref, b2_ref, out_ref, h_ref):
    @pl.when(pl.program_id(0) == 0)
    def _():
        h = jnp.dot(emb_ref[...], w1_ref[...],
                    preferred_element_type=jnp.float32)
        h_ref[...] = jnp.maximum(h + b1_ref[...], 0.0)

    out_ref[...] = (
        jnp.dot(h_ref[...], w2_ref[...], preferred_element_type=jnp.float32)
        + b2_ref[...]
    )


def kernel(x, table, W1, b1, W2, b2):
    B = x.shape[0]
    V, D = table.shape
    H = W1.shape[1]
    NV = W2.shape[1]
    vt = _VOCAB_TILE
    grid = (NV + vt - 1) // vt

    emb = _make_sc_gather(V, D, B)(table, x.astype(jnp.int32))

    out = pl.pallas_call(
        _mlp_body,
        grid=(grid,),
        in_specs=[
            pl.BlockSpec((B, D), lambda j: (0, 0)),
            pl.BlockSpec((D, H), lambda j: (0, 0)),
            pl.BlockSpec((1, H), lambda j: (0, 0)),
            pl.BlockSpec((H, vt), lambda j: (0, j)),
            pl.BlockSpec((1, vt), lambda j: (0, j)),
        ],
        out_specs=pl.BlockSpec((B, vt), lambda j: (0, j)),
        out_shape=jax.ShapeDtypeStruct((B, NV), jnp.float32),
        scratch_shapes=[pltpu.VMEM((B, H), jnp.float32)],
    )(emb, W1, b1.reshape(1, H), W2, b2.reshape(1, NV))

    return out


# speedup vs baseline: 2.8247x; 2.8247x over previous
import functools

import jax
import jax.numpy as jnp
from jax import lax
from jax.experimental import pallas as pl
from jax.experimental.pallas import tpu as pltpu
from jax.experimental.pallas import tpu_sc as plsc

_VOCAB_TILE = 2048


def _make_sc_gather(V, D, B):
    info = plsc.get_sparse_core_info()
    NC, NS = info.num_cores, info.num_subcores
    NW = NC * NS
    assert D % info.num_lanes == 0 and B % (8 * NW) == 0
    b_per_w = B // NW
    mesh = plsc.VectorSubcoreMesh(core_axis_name="c", subcore_axis_name="s")

    @functools.partial(
        pl.kernel,
        mesh=mesh,
        out_type=jax.ShapeDtypeStruct((B, D), jnp.float32),
        scratch_types=[
            pltpu.VMEM((b_per_w,), jnp.int32),
            pltpu.VMEM((b_per_w, D), jnp.float32),
            pltpu.SemaphoreType.DMA,
        ],
        compiler_params=pltpu.CompilerParams(use_tc_tiling_on_sc=False),
    )
    def gather(table_hbm, idx_hbm, out_hbm, idx_v, rows_v, sem):
        wid = lax.axis_index("s") * NC + lax.axis_index("c")
        base = wid * b_per_w
        pltpu.sync_copy(idx_hbm.at[pl.ds(base, b_per_w)], idx_v)
        pltpu.async_copy(table_hbm.at[idx_v], rows_v, sem).wait()
        pltpu.sync_copy(rows_v, out_hbm.at[pl.ds(base, b_per_w)])

    return gather


def _mlp_body(emb_ref, w1_ref, b1_ref, w2t_ref, b2_ref, out_ref, ht_ref):
    @pl.when(pl.program_id(0) == 0)
    def _():
        ht = lax.dot_general(
            w1_ref[...], emb_ref[...],
            dimension_numbers=(((0,), (1,)), ((), ())),
            preferred_element_type=jnp.float32,
        )
        ht_ref[...] = jnp.maximum(ht + b1_ref[...].T, 0.0)

    out_ref[...] = (
        jnp.dot(w2t_ref[...], ht_ref[...], preferred_element_type=jnp.float32)
        + b2_ref[...].T
    )


def kernel(x, table, W1, b1, W2, b2):
    B = x.shape[0]
    V, D = table.shape
    H = W1.shape[1]
    NV = W2.shape[1]
    vt = _VOCAB_TILE
    grid = (NV + vt - 1) // vt

    emb = _make_sc_gather(V, D, B)(table, x.astype(jnp.int32))

    out_t = pl.pallas_call(
        _mlp_body,
        grid=(grid,),
        in_specs=[
            pl.BlockSpec((B, D), lambda j: (0, 0)),
            pl.BlockSpec((D, H), lambda j: (0, 0)),
            pl.BlockSpec((1, H), lambda j: (0, 0)),
            pl.BlockSpec((vt, H), lambda j: (j, 0)),
            pl.BlockSpec((1, vt), lambda j: (0, j)),
        ],
        out_specs=pl.BlockSpec((vt, B), lambda j: (j, 0)),
        out_shape=jax.ShapeDtypeStruct((NV, B), jnp.float32),
        scratch_shapes=[pltpu.VMEM((H, B), jnp.float32)],
    )(emb, W1, b1.reshape(1, H), W2.T, b2.reshape(1, NV))

    return out_t.T


# speedup vs baseline: 3.1887x; 1.1289x over previous
import functools

import jax
import jax.numpy as jnp
from jax import lax
from jax.experimental import pallas as pl
from jax.experimental.pallas import tpu as pltpu
from jax.experimental.pallas import tpu_sc as plsc

_VOCAB_TILE = 2048


def _make_sc_gather(NT, D, B):
    info = plsc.get_sparse_core_info()
    NC, NS = info.num_cores, info.num_subcores
    NW = NC * NS
    assert D % info.num_lanes == 0 and B % (8 * NW) == 0
    b_per_w = B // NW
    mesh = plsc.VectorSubcoreMesh(core_axis_name="c", subcore_axis_name="s")

    @functools.partial(
        pl.kernel,
        mesh=mesh,
        out_type=jax.ShapeDtypeStruct((B, 8, D), jnp.float32),
        scratch_types=[
            pltpu.VMEM((b_per_w,), jnp.int32),
            pltpu.VMEM((b_per_w, 8, D), jnp.float32),
            pltpu.SemaphoreType.DMA,
        ],
        compiler_params=pltpu.CompilerParams(
            use_tc_tiling_on_sc=True, needs_layout_passes=False),
    )
    def gather(table_hbm, tidx_hbm, out_hbm, idx_v, rows_v, sem):
        wid = lax.axis_index("s") * NC + lax.axis_index("c")
        base = wid * b_per_w
        pltpu.sync_copy(tidx_hbm.at[pl.ds(base, b_per_w)], idx_v)
        lanes = lax.iota(jnp.int32, 16)
        copies = []
        for c in range(b_per_w // 16):
            chunk = idx_v[pl.ds(c * 16, 16)]
            for l in range(16):
                ti = lax.reduce_sum_p.bind(
                    jnp.where(lanes == l, chunk, 0), axes=(0,))
                copies.append(pltpu.async_copy(
                    table_hbm.at[pl.ds(ti, 1)],
                    rows_v.at[pl.ds(c * 16 + l, 1)], sem))
        for cp in copies:
            cp.wait()
        pltpu.sync_copy(rows_v, out_hbm.at[pl.ds(base, b_per_w)])

    return gather


def _mlp_body(embw_ref, s2_ref, w1_ref, b1_ref, w2t_ref, b2_ref,
              out_ref, ht_ref):
    @pl.when(pl.program_id(0) == 0)
    def _():
        sel = s2_ref[...] == lax.broadcasted_iota(jnp.int32, (1, 8), 1)
        emb = jnp.zeros(
            (embw_ref.shape[0], embw_ref.shape[2]), jnp.float32)
        for s in range(8):
            emb = emb + embw_ref[:, s, :] * jnp.where(
                sel[:, s:s + 1], 1.0, 0.0)
        ht = lax.dot_general(
            w1_ref[...], emb,
            dimension_numbers=(((0,), (1,)), ((), ())),
            preferred_element_type=jnp.float32,
        )
        ht_ref[...] = jnp.maximum(ht + b1_ref[...].T, 0.0)

    out_ref[...] = (
        jnp.dot(w2t_ref[...], ht_ref[...], preferred_element_type=jnp.float32)
        + b2_ref[...].T
    )


def kernel(x, table, W1, b1, W2, b2):
    B = x.shape[0]
    V, D = table.shape
    H = W1.shape[1]
    NV = W2.shape[1]
    vt = _VOCAB_TILE
    grid = (NV + vt - 1) // vt

    xi = x.astype(jnp.int32)
    table3 = table.reshape(V // 8, 8, D)
    embw = _make_sc_gather(V // 8, D, B)(table3, xi // 8)
    s2 = (xi % 8).reshape(B, 1)

    out_t = pl.pallas_call(
        _mlp_body,
        grid=(grid,),
        in_specs=[
            pl.BlockSpec((B, 8, D), lambda j: (0, 0, 0)),
            pl.BlockSpec((B, 1), lambda j: (0, 0)),
            pl.BlockSpec((D, H), lambda j: (0, 0)),
            pl.BlockSpec((1, H), lambda j: (0, 0)),
            pl.BlockSpec((vt, H), lambda j: (j, 0)),
            pl.BlockSpec((1, vt), lambda j: (0, j)),
        ],
        out_specs=pl.BlockSpec((vt, B), lambda j: (j, 0)),
        out_shape=jax.ShapeDtypeStruct((NV, B), jnp.float32),
        scratch_shapes=[pltpu.VMEM((H, B), jnp.float32)],
    )(embw, s2, W1, b1.reshape(1, H), W2.T, b2.reshape(1, NV))

    return out_t.T


# speedup vs baseline: 3.2603x; 1.0224x over previous
import functools

import jax
import jax.numpy as jnp
from jax import lax
from jax.experimental import pallas as pl
from jax.experimental.pallas import tpu as pltpu
from jax.experimental.pallas import tpu_sc as plsc

_VOCAB_TILE = 4096


def _make_sc_gather(NT, D, B):
    info = plsc.get_sparse_core_info()
    NC, NS = info.num_cores, info.num_subcores
    NW = NC * NS
    assert D % info.num_lanes == 0 and B % (8 * NW) == 0
    b_per_w = B // NW
    mesh = plsc.VectorSubcoreMesh(core_axis_name="c", subcore_axis_name="s")

    @functools.partial(
        pl.kernel,
        mesh=mesh,
        out_type=jax.ShapeDtypeStruct((B, 8, D), jnp.float32),
        scratch_types=[
            pltpu.VMEM((b_per_w,), jnp.int32),
            pltpu.VMEM((b_per_w, 8, D), jnp.float32),
            pltpu.SemaphoreType.DMA,
        ],
        compiler_params=pltpu.CompilerParams(
            use_tc_tiling_on_sc=True, needs_layout_passes=False),
    )
    def gather(table_hbm, tidx_hbm, out_hbm, idx_v, rows_v, sem):
        wid = lax.axis_index("s") * NC + lax.axis_index("c")
        base = wid * b_per_w
        pltpu.sync_copy(tidx_hbm.at[pl.ds(base, b_per_w)], idx_v)
        lanes = lax.iota(jnp.int32, 16)
        copies = []
        for c in range(b_per_w // 16):
            chunk = idx_v[pl.ds(c * 16, 16)]
            for l in range(16):
                ti = lax.reduce_sum_p.bind(
                    jnp.where(lanes == l, chunk, 0), axes=(0,))
                copies.append(pltpu.async_copy(
                    table_hbm.at[pl.ds(ti, 1)],
                    rows_v.at[pl.ds(c * 16 + l, 1)], sem))
        for cp in copies:
            cp.wait()
        pltpu.sync_copy(rows_v, out_hbm.at[pl.ds(base, b_per_w)])

    return gather


def _mlp_body(embw_ref, s2_ref, w1_ref, b1_ref, w2t_ref, b2_ref,
              out_ref, ht_ref):
    @pl.when(pl.program_id(0) == 0)
    def _():
        sel = s2_ref[...] == lax.broadcasted_iota(jnp.int32, (1, 8), 1)
        emb = jnp.zeros(
            (embw_ref.shape[0], embw_ref.shape[2]), jnp.float32)
        for s in range(8):
            emb = emb + embw_ref[:, s, :] * jnp.where(
                sel[:, s:s + 1], 1.0, 0.0)
        ht = lax.dot_general(
            w1_ref[...], emb,
            dimension_numbers=(((0,), (1,)), ((), ())),
            preferred_element_type=jnp.float32,
        )
        ht_ref[...] = jnp.maximum(ht + b1_ref[...].T, 0.0)

    out_ref[...] = (
        jnp.dot(w2t_ref[...], ht_ref[...], preferred_element_type=jnp.float32)
        + b2_ref[...].T
    )


def kernel(x, table, W1, b1, W2, b2):
    B = x.shape[0]
    V, D = table.shape
    H = W1.shape[1]
    NV = W2.shape[1]
    vt = _VOCAB_TILE
    grid = (NV + vt - 1) // vt

    xi = x.astype(jnp.int32)
    table3 = table.reshape(V // 8, 8, D)
    embw = _make_sc_gather(V // 8, D, B)(table3, xi // 8)
    s2 = (xi % 8).reshape(B, 1)

    out_t = pl.pallas_call(
        _mlp_body,
        grid=(grid,),
        in_specs=[
            pl.BlockSpec((B, 8, D), lambda j: (0, 0, 0)),
            pl.BlockSpec((B, 1), lambda j: (0, 0)),
            pl.BlockSpec((D, H), lambda j: (0, 0)),
            pl.BlockSpec((1, H), lambda j: (0, 0)),
            pl.BlockSpec((vt, H), lambda j: (j, 0)),
            pl.BlockSpec((1, vt), lambda j: (0, j)),
        ],
        out_specs=pl.BlockSpec((vt, B), lambda j: (j, 0)),
        out_shape=jax.ShapeDtypeStruct((NV, B), jnp.float32),
        scratch_shapes=[pltpu.VMEM((H, B), jnp.float32)],
    )(embw, s2, W1, b1.reshape(1, H), W2.T, b2.reshape(1, NV))

    return out_t.T


# speedup vs baseline: 3.5379x; 1.0851x over previous
import functools

import jax
import jax.numpy as jnp
from jax import lax
from jax.experimental import pallas as pl
from jax.experimental.pallas import tpu as pltpu
from jax.experimental.pallas import tpu_sc as plsc

_VOCAB_TILE = 4096
_LANES = 16


def _extract_scalar(chunk, lanes, l):
    return lax.reduce_sum_p.bind(
        jnp.where(lanes == l, chunk, 0), axes=(0,))


def _make_sc_gather(D, B):
    info = plsc.get_sparse_core_info()
    NC, NS = info.num_cores, info.num_subcores
    NW = NC * NS
    assert D % info.num_lanes == 0 and B % (8 * NW) == 0
    b_per_w = B // NW
    half = b_per_w // 2
    ngrp = D // 8
    mesh = plsc.VectorSubcoreMesh(core_axis_name="c", subcore_axis_name="s")

    @functools.partial(
        pl.kernel,
        mesh=mesh,
        out_type=jax.ShapeDtypeStruct((B, D), jnp.float32),
        scratch_types=[
            pltpu.VMEM((b_per_w,), jnp.int32),
            pltpu.VMEM((half, ngrp, 8, 128), jnp.float32),
            pltpu.VMEM((b_per_w, D), jnp.float32),
            pltpu.SemaphoreType.DMA,
        ],
        compiler_params=pltpu.CompilerParams(
            use_tc_tiling_on_sc=True, needs_layout_passes=False),
    )
    def gather(tt_hbm, x_hbm, out_hbm, idx_v, tiles_v, obuf, sem):
        wid = lax.axis_index("s") * NC + lax.axis_index("c")
        base = wid * b_per_w
        pltpu.sync_copy(x_hbm.at[pl.ds(base, b_per_w)], idx_v)
        lanes = lax.iota(jnp.int32, _LANES)
        gv0 = lanes // 8
        sv0 = lanes % 8
        for hf in range(2):
            copies = []
            scalars = []
            for i in range(half):
                e = hf * half + i
                chunk = idx_v[pl.ds((e // _LANES) * _LANES, _LANES)]
                xe = _extract_scalar(chunk, lanes, e % _LANES)
                ti = pl.multiple_of((xe // 128) * 128, 128)
                scalars.append(xe)
                for g in range(ngrp):
                    copies.append(pltpu.async_copy(
                        tt_hbm.at[pl.ds(g * 8, 8), pl.ds(ti, 128)],
                        tiles_v.at[i, g], sem))
            for cp in copies:
                cp.wait()
            for i in range(half):
                e = hf * half + i
                li = lax.rem(scalars[i], 128)
                iv = jnp.full((_LANES,), i, jnp.int32)
                lv = jnp.full((_LANES,), li, jnp.int32)
                for c in range(D // _LANES):
                    vals = plsc.load_gather(
                        tiles_v,
                        [iv, gv0 + c * (_LANES // 8), sv0, lv])
                    obuf[e, pl.ds(c * _LANES, _LANES)] = vals
        pltpu.sync_copy(obuf, out_hbm.at[pl.ds(base, b_per_w)])

    return gather


def _mlp_body(emb_ref, w1_ref, b1_ref, w2t_ref, b2_ref, out_ref, ht_ref):
    @pl.when(pl.program_id(0) == 0)
    def _():
        ht = lax.dot_general(
            w1_ref[...], emb_ref[...],
            dimension_numbers=(((0,), (1,)), ((), ())),
            preferred_element_type=jnp.float32,
        )
        ht_ref[...] = jnp.maximum(ht + b1_ref[...].T, 0.0)

    out_ref[...] = (
        jnp.dot(w2t_ref[...], ht_ref[...], preferred_element_type=jnp.float32)
        + b2_ref[...].T
    )


def kernel(x, table, W1, b1, W2, b2):
    B = x.shape[0]
    V, D = table.shape
    H = W1.shape[1]
    NV = W2.shape[1]
    vt = _VOCAB_TILE
    grid = (NV + vt - 1) // vt

    xi = x.astype(jnp.int32)
    emb = _make_sc_gather(D, B)(table.T, xi)

    out_t = pl.pallas_call(
        _mlp_body,
        grid=(grid,),
        in_specs=[
            pl.BlockSpec((B, D), lambda j: (0, 0)),
            pl.BlockSpec((D, H), lambda j: (0, 0)),
            pl.BlockSpec((1, H), lambda j: (0, 0)),
            pl.BlockSpec((vt, H), lambda j: (j, 0)),
            pl.BlockSpec((1, vt), lambda j: (0, j)),
        ],
        out_specs=pl.BlockSpec((vt, B), lambda j: (j, 0)),
        out_shape=jax.ShapeDtypeStruct((NV, B), jnp.float32),
        scratch_shapes=[pltpu.VMEM((H, B), jnp.float32)],
    )(emb, W1, b1.reshape(1, H), W2.T, b2.reshape(1, NV))

    return out_t.T


# speedup vs baseline: 3.5753x; 1.0106x over previous
import functools

import jax
import jax.numpy as jnp
from jax import lax
from jax.experimental import pallas as pl
from jax.experimental.pallas import tpu as pltpu
from jax.experimental.pallas import tpu_sc as plsc

_VOCAB_TILE = 4096
_LANES = 16


def _extract_scalar(chunk, lanes, l):
    return lax.reduce_sum_p.bind(
        jnp.where(lanes == l, chunk, 0), axes=(0,))


def _make_sc_gather(D, B):
    info = plsc.get_sparse_core_info()
    NC, NS = info.num_cores, info.num_subcores
    NW = NC * NS
    assert D % info.num_lanes == 0 and B % (8 * NW) == 0
    b_per_w = B // NW
    half = b_per_w // 2
    ngrp = D // 8
    mesh = plsc.VectorSubcoreMesh(core_axis_name="c", subcore_axis_name="s")

    @functools.partial(
        pl.kernel,
        mesh=mesh,
        out_type=jax.ShapeDtypeStruct((B, D), jnp.float32),
        scratch_types=[
            pltpu.VMEM((b_per_w,), jnp.int32),
            pltpu.VMEM((2, b_per_w // 4, D, 128), jnp.float32),
            pltpu.VMEM((b_per_w, D), jnp.float32),
            pltpu.SemaphoreType.DMA,
            pltpu.SemaphoreType.DMA,
        ],
        compiler_params=pltpu.CompilerParams(
            use_tc_tiling_on_sc=True, needs_layout_passes=False),
    )
    def gather(tt_hbm, x_hbm, out_hbm, idx_v, tiles_v, obuf, sem0, sem1):
        wid = lax.axis_index("s") * NC + lax.axis_index("c")
        base = wid * b_per_w
        q = b_per_w // 4
        pltpu.sync_copy(x_hbm.at[pl.ds(base, b_per_w)], idx_v)
        lanes = lax.iota(jnp.int32, _LANES)
        sems = [sem0, sem1]

        def fire(qi):
            handles, scalars = [], []
            for i in range(q):
                e = qi * q + i
                chunk = idx_v[pl.ds((e // _LANES) * _LANES, _LANES)]
                xe = _extract_scalar(chunk, lanes, e % _LANES)
                ti = pl.multiple_of((xe // 128) * 128, 128)
                scalars.append(xe)
                handles.append(pltpu.async_copy(
                    tt_hbm.at[:, pl.ds(ti, 128)],
                    tiles_v.at[qi % 2, i], sems[qi % 2]))
            return handles, scalars

        def drain_extract(qi, handles, scalars):
            for h in handles:
                h.wait()
            for i in range(q):
                e = qi * q + i
                li = lax.rem(scalars[i], 128)
                iv = jnp.full((_LANES,), i, jnp.int32)
                hv = jnp.full((_LANES,), qi % 2, jnp.int32)
                lv = jnp.full((_LANES,), li, jnp.int32)
                for c in range(D // _LANES):
                    vals = plsc.load_gather(
                        tiles_v,
                        [hv, iv, lanes + c * _LANES, lv])
                    obuf[e, pl.ds(c * _LANES, _LANES)] = vals

        inflight = [fire(0), fire(1)]
        for qi in range(4):
            handles, scalars = inflight[qi % 2]
            drain_extract(qi, handles, scalars)
            if qi + 2 < 4:
                inflight[qi % 2] = fire(qi + 2)
        pltpu.sync_copy(obuf, out_hbm.at[pl.ds(base, b_per_w)])

    return gather


def _mlp_body(emb_ref, w1_ref, b1_ref, w2t_ref, b2_ref, out_ref, ht_ref):
    @pl.when(pl.program_id(0) == 0)
    def _():
        ht = lax.dot_general(
            w1_ref[...], emb_ref[...],
            dimension_numbers=(((0,), (1,)), ((), ())),
            preferred_element_type=jnp.float32,
        )
        ht_ref[...] = jnp.maximum(ht + b1_ref[...].T, 0.0)

    out_ref[...] = (
        jnp.dot(w2t_ref[...], ht_ref[...], preferred_element_type=jnp.float32)
        + b2_ref[...].T
    )


def kernel(x, table, W1, b1, W2, b2):
    B = x.shape[0]
    V, D = table.shape
    H = W1.shape[1]
    NV = W2.shape[1]
    vt = _VOCAB_TILE
    grid = (NV + vt - 1) // vt

    xi = x.astype(jnp.int32)
    emb = _make_sc_gather(D, B)(table.T, xi)

    out_t = pl.pallas_call(
        _mlp_body,
        grid=(grid,),
        in_specs=[
            pl.BlockSpec((B, D), lambda j: (0, 0)),
            pl.BlockSpec((D, H), lambda j: (0, 0)),
            pl.BlockSpec((1, H), lambda j: (0, 0)),
            pl.BlockSpec((vt, H), lambda j: (j, 0)),
            pl.BlockSpec((1, vt), lambda j: (0, j)),
        ],
        out_specs=pl.BlockSpec((vt, B), lambda j: (j, 0)),
        out_shape=jax.ShapeDtypeStruct((NV, B), jnp.float32),
        scratch_shapes=[pltpu.VMEM((H, B), jnp.float32)],
    )(emb, W1, b1.reshape(1, H), W2.T, b2.reshape(1, NV))

    return out_t.T
